# Initial kernel scaffold; baseline (speedup 1.0000x reference)
import jax
import jax.numpy as jnp
from jax.experimental import pallas as pl


def kernel(users, feats, user_table, item_table):
    raise NotImplementedError("write your pallas kernel here")


# speedup vs baseline: 1.6808x; 1.6808x over previous
import functools

import jax
import jax.numpy as jnp
from jax import lax
from jax.experimental import pallas as pl
from jax.experimental.pallas import tpu as pltpu
from jax.experimental.pallas import tpu_sc as plsc

_B = 16384
_NCAND = 21
_D = 64
_F = _B * _NCAND

_INFO = plsc.get_sparse_core_info()
_NC = _INFO.num_cores
_NS = _INFO.num_subcores
_NW = _NC * _NS
_UPW = _B // _NW
_FPW = _F // _NW
_CHUNK = 896
_NCHUNK = _FPW // _CHUNK


def _tower_body(users_hbm, feats_hbm, utab_hbm, itab_hbm,
                uout_hbm, fout_hbm,
                idx_v, rows0, rows1, gsem0, gsem1):
    rows = (rows0, rows1)
    gsem = (gsem0, gsem1)
    wid = lax.axis_index("s") * _NC + lax.axis_index("c")
    ubase = wid * _UPW
    fbase = wid * _FPW

    pltpu.sync_copy(users_hbm.at[pl.ds(ubase, _UPW)], idx_v.at[pl.ds(0, _UPW)])
    pltpu.async_copy(utab_hbm.at[idx_v.at[pl.ds(0, _UPW)]],
                     rows0.at[pl.ds(0, _UPW)], gsem0).wait()
    pltpu.sync_copy(rows0.at[pl.ds(0, _UPW)], uout_hbm.at[pl.ds(ubase, _UPW)])

    pltpu.sync_copy(feats_hbm.at[pl.ds(fbase, _FPW)], idx_v)
    copies = [None, None]
    for c in range(_NCHUNK):
        b = c & 1
        copies[b] = pltpu.async_copy(
            itab_hbm.at[idx_v.at[pl.ds(c * _CHUNK, _CHUNK)]], rows[b], gsem[b])
        if c >= 1:
            pb = (c - 1) & 1
            copies[pb].wait()
            pltpu.sync_copy(rows[pb],
                            fout_hbm.at[pl.ds(fbase + (c - 1) * _CHUNK, _CHUNK)])
    lb = (_NCHUNK - 1) & 1
    copies[lb].wait()
    pltpu.sync_copy(rows[lb],
                    fout_hbm.at[pl.ds(fbase + (_NCHUNK - 1) * _CHUNK, _CHUNK)])


@jax.jit
def _tower_sc(users, feats_flat, user_table, item_table):
    mesh = plsc.VectorSubcoreMesh(core_axis_name="c", subcore_axis_name="s")
    return pl.kernel(
        _tower_body,
        out_type=(jax.ShapeDtypeStruct((_B, _D), jnp.float32),
                  jax.ShapeDtypeStruct((_F, _D), jnp.float32)),
        mesh=mesh,
        compiler_params=pltpu.CompilerParams(use_tc_tiling_on_sc=False),
        scratch_types=[
            pltpu.VMEM((_FPW,), jnp.int32),
            pltpu.VMEM((_CHUNK, _D), jnp.float32),
            pltpu.VMEM((_CHUNK, _D), jnp.float32),
            pltpu.SemaphoreType.DMA,
            pltpu.SemaphoreType.DMA,
        ],
    )(users, feats_flat, user_table, item_table)


def kernel(users, feats, user_table, item_table):
    user_emb, feat_flat = _tower_sc(users, feats.reshape(_F), user_table,
                                    item_table)
    return (user_emb, feat_flat.reshape(_B, _NCAND, _D))
